# baseline (speedup 1.0000x reference)
import functools

import numpy as np
import jax
import jax.numpy as jnp
from jax import lax
from jax.experimental import pallas as pl
from jax.experimental.pallas import tpu as pltpu
from jax.experimental.pallas import tpu_sc as plsc

_NUM_CLASSES = 599
_OUT_HW = 40
_IN_HW = 64
_PAD = 48
_B = 8
_N = 100
_C = 256
_NW = 32
_NPW = _N * _B // _NW
_IDXW = 112


def _build_ap() -> np.ndarray:
    s = (np.arange(_OUT_HW, dtype=np.float64) + 0.5) * (_IN_HW / _OUT_HW) - 0.5
    h0 = np.floor(s).astype(np.int64)
    frac = s - h0
    r = np.zeros((_OUT_HW, _IN_HW), dtype=np.float64)
    r[np.arange(_OUT_HW), h0] = 1.0 - frac
    r[np.arange(_OUT_HW), h0 + 1] = frac
    ap = np.zeros((_PAD, _IN_HW), dtype=np.float64)
    ap[1:_OUT_HW + 1] = np.cumsum(r, axis=0)
    return ap.astype(np.float32)


_AP = _build_ap()


def _matmul_kernel(a_ref, x_ref, o_ref):
    o_ref[0] = jnp.dot(a_ref[...], x_ref[0],
                       precision=lax.Precision.HIGHEST,
                       preferred_element_type=jnp.float32)


def _apply_ap(x, ap):
    b, k, m = x.shape
    return pl.pallas_call(
        _matmul_kernel,
        grid=(b,),
        in_specs=[
            pl.BlockSpec((_PAD, _IN_HW), lambda i: (0, 0)),
            pl.BlockSpec((1, k, m), lambda i: (i, 0, 0)),
        ],
        out_specs=pl.BlockSpec((1, _PAD, m), lambda i: (i, 0, 0)),
        out_shape=jax.ShapeDtypeStruct((b, _PAD, m), jnp.float32),
    )(ap, x)


def _gather_body(tab_hbm, idx_hbm, out_hbm, idx_v, rows_v, pooled_v, sem):
    wid = lax.axis_index("s") * 2 + lax.axis_index("c")
    pltpu.sync_copy(idx_hbm.at[wid], idx_v)
    pltpu.async_copy(tab_hbm.at[idx_v], rows_v, sem).wait()

    def body(i, carry):
        for j in range(_C // 16):
            sl = pl.ds(j * 16, 16)
            v = (rows_v[4 * i, sl] - rows_v[4 * i + 1, sl]
                 - rows_v[4 * i + 2, sl] + rows_v[4 * i + 3, sl])
            pooled_v[i, sl] = v
        return carry

    lax.fori_loop(0, _NPW, body, 0)
    pltpu.sync_copy(pooled_v, out_hbm.at[wid])


@functools.cache
def _gather_call():
    return functools.partial(
        pl.kernel,
        out_type=jax.ShapeDtypeStruct((_NW, _NPW, _C), jnp.float32),
        mesh=plsc.VectorSubcoreMesh(core_axis_name="c", subcore_axis_name="s",
                                    num_cores=2, num_subcores=16),
        scratch_types=[
            pltpu.VMEM((_IDXW,), jnp.int32),
            pltpu.VMEM((_IDXW, _C), jnp.float32),
            pltpu.VMEM((_NPW, _C), jnp.float32),
            pltpu.SemaphoreType.DMA,
        ],
    )(_gather_body)


def _segment_kernel(pooled_ref, boxes_ref, cls_ref, o_ref):
    boxes = boxes_ref[0]
    rb = jnp.round(boxes * jnp.float32(_OUT_HW / 1024.0))
    x1 = jnp.maximum(rb[:, 0:1], 0.0)
    y1 = jnp.maximum(rb[:, 1:2], 0.0)
    x2 = jnp.minimum(rb[:, 2:3], float(_OUT_HW))
    y2 = jnp.minimum(rb[:, 3:4], float(_OUT_HW))
    valid = (x1 < x2) & (y1 < y2)
    area = (x2 - x1) * (y2 - y1)
    w = jnp.where(valid, 1.0 / jnp.maximum(area, 1.0), 0.0)
    wp = pooled_ref[0] * w
    colv = valid.astype(jnp.float32)
    cls = cls_ref[0]
    ks = lax.broadcasted_iota(jnp.int32, (640, _N), 0)
    m = (ks == cls).astype(jnp.float32)
    sums = jnp.dot(m, wp, precision=lax.Precision.HIGHEST,
                   preferred_element_type=jnp.float32)
    cnt = jnp.dot(m, colv, precision=lax.Precision.HIGHEST,
                  preferred_element_type=jnp.float32)
    o_ref[0] = sums * (1.0 / jnp.maximum(cnt, 1.0))


def kernel(features, gt_boxes, gt_classes):
    b, c, h, w0 = features.shape
    ap = jnp.asarray(_AP)

    ft = jnp.transpose(features, (0, 2, 3, 1)).reshape(b, h, w0 * c)
    u = _apply_ap(ft, ap)
    ut = (u.reshape(b, _PAD, w0, c).transpose(0, 2, 1, 3)
          .reshape(b, w0, _PAD * c))
    s = _apply_ap(ut, ap)
    tab = s.reshape(b * _PAD * _PAD, c)

    rb = jnp.round(gt_boxes * jnp.float32(_OUT_HW / 1024.0))
    x1 = jnp.clip(rb[..., 0], 0.0, float(_PAD - 1)).astype(jnp.int32)
    y1 = jnp.clip(rb[..., 1], 0.0, float(_PAD - 1)).astype(jnp.int32)
    x2 = jnp.clip(rb[..., 2], 0.0, float(_PAD - 1)).astype(jnp.int32)
    y2 = jnp.clip(rb[..., 3], 0.0, float(_PAD - 1)).astype(jnp.int32)
    base = (jnp.arange(b, dtype=jnp.int32) * (_PAD * _PAD))[:, None]
    corners = jnp.stack([
        base + x2 * _PAD + y2,
        base + x1 * _PAD + y2,
        base + x2 * _PAD + y1,
        base + x1 * _PAD + y1,
    ], axis=-1)
    idx = corners.reshape(_NW, 4 * _NPW)
    idx = jnp.pad(idx, ((0, 0), (0, _IDXW - 4 * _NPW)))

    pooled = _gather_call()(tab, idx).reshape(b, _N, c)

    cls3 = gt_classes.astype(jnp.int32).reshape(b, 1, _N)
    out = pl.pallas_call(
        _segment_kernel,
        grid=(b,),
        in_specs=[
            pl.BlockSpec((1, _N, c), lambda i: (i, 0, 0)),
            pl.BlockSpec((1, _N, 4), lambda i: (i, 0, 0)),
            pl.BlockSpec((1, 1, _N), lambda i: (i, 0, 0)),
        ],
        out_specs=pl.BlockSpec((1, 640, c), lambda i: (i, 0, 0)),
        out_shape=jax.ShapeDtypeStruct((b, 640, c), jnp.float32),
    )(pooled, gt_boxes, cls3)
    return out[:, :_NUM_CLASSES, :]
